# baseline (speedup 1.0000x reference)
import jax
import jax.numpy as jnp
from jax import lax
from jax.experimental import pallas as pl
from jax.experimental.pallas import tpu as pltpu

_N_WORDS = 100000
_DIM = 16
_CNT_ROWS = 782
_CPAD = _CNT_ROWS * 128 - _N_WORDS


def _body(ix_ref, vec_in, sum_in, cnt_in, cov_in, sum_out, cnt_out, cov_out):
    ix = ix_ref[0]
    vec = vec_in[...]

    r = (ix // 8) % 8
    g = ix % 8
    row_i = lax.broadcasted_iota(jnp.int32, (8, 128), 0)
    lane_i = lax.broadcasted_iota(jnp.int32, (8, 128), 1)
    vec_t8 = jnp.broadcast_to(jnp.concatenate([vec] * 8, axis=1), (8, 128))
    hit_s = jnp.logical_and(row_i == r, lane_i // _DIM == g)
    sum_out[...] = sum_in[...] + jnp.where(hit_s, vec_t8, 0.0)

    r2 = (ix // 128) % 8
    c2 = ix % 128
    hit_c = jnp.logical_and(row_i == r2, lane_i == c2)
    cnt_out[...] = cnt_in[...] + hit_c.astype(jnp.int32)

    iota_j = lax.broadcasted_iota(jnp.int32, (_DIM, 256), 0)
    iota_l = lax.broadcasted_iota(jnp.int32, (_DIM, 256), 1)
    m_int = (iota_l // _DIM == iota_j).astype(jnp.float32)
    b = lax.dot_general(vec, m_int, (((1,), (0,)), ((), ())),
                        precision=lax.Precision.HIGHEST,
                        preferred_element_type=jnp.float32)
    a = jnp.concatenate([vec] * _DIM, axis=1)
    outer_flat = a * b
    r3 = ix % 8
    row_i2 = lax.broadcasted_iota(jnp.int32, (8, 256), 0)
    outer_b = jnp.broadcast_to(outer_flat, (8, 256))
    cov_out[...] = cov_in[...] + jnp.where(row_i2 == r3, outer_b, 0.0)


def kernel(ix, vec, sum_buf, counts, cov_buf):
    ix_arr = jnp.reshape(jnp.asarray(ix, jnp.int32), (1,))
    cpad = jnp.concatenate(
        [counts, jnp.zeros((_CPAD,), jnp.int32)]).reshape(_CNT_ROWS, 128)
    zf = lax.optimization_barrier(jnp.zeros((), jnp.float32))
    sum_t = sum_buf.reshape(_N_WORDS // 8, 128) + zf
    cov_t = cov_buf.reshape(_N_WORDS, 256) + zf
    grid_spec = pltpu.PrefetchScalarGridSpec(
        num_scalar_prefetch=1,
        grid=(1,),
        in_specs=[
            pl.BlockSpec((1, _DIM), lambda i, s: (0, 0)),
            pl.BlockSpec((8, 128), lambda i, s: (s[0] // 64, 0)),
            pl.BlockSpec((8, 128), lambda i, s: (s[0] // 1024, 0)),
            pl.BlockSpec((8, 256), lambda i, s: (s[0] // 8, 0)),
        ],
        out_specs=[
            pl.BlockSpec((8, 128), lambda i, s: (s[0] // 64, 0)),
            pl.BlockSpec((8, 128), lambda i, s: (s[0] // 1024, 0)),
            pl.BlockSpec((8, 256), lambda i, s: (s[0] // 8, 0)),
        ],
    )
    out = pl.pallas_call(
        _body,
        grid_spec=grid_spec,
        out_shape=[
            jax.ShapeDtypeStruct((_N_WORDS // 8, 128), jnp.float32),
            jax.ShapeDtypeStruct((_CNT_ROWS, 128), jnp.int32),
            jax.ShapeDtypeStruct((_N_WORDS, 256), jnp.float32),
        ],
        input_output_aliases={2: 0, 3: 1, 4: 2},
    )(ix_arr, vec.reshape(1, _DIM), sum_t, cpad, cov_t)
    return (out[0].reshape(_N_WORDS, _DIM),
            out[1].reshape(-1)[:_N_WORDS],
            out[2].reshape(_N_WORDS, _DIM, _DIM))


# speedup vs baseline: 4.5694x; 4.5694x over previous
import jax
import jax.numpy as jnp
from jax import lax
from jax.experimental import pallas as pl
from jax.experimental.pallas import tpu as pltpu

_N_WORDS = 100000
_DIM = 16
_CNT_ROWS = 782
_CPAD = _CNT_ROWS * 128 - _N_WORDS


def _body(ix_ref, vecb_in, vhi_in, vlo_in, sum_in, cnt_in, cov_in,
          sum_out, cnt_out, cov_out):
    ix = ix_ref[0]
    c = ix % 128

    lane16 = lax.broadcasted_iota(jnp.int32, (_DIM, 128), 1)
    sum_out[...] = sum_in[...] + jnp.where(lane16 == c, vecb_in[...], 0.0)

    r2 = (ix // 128) % 8
    row8 = lax.broadcasted_iota(jnp.int32, (8, 128), 0)
    lane8 = lax.broadcasted_iota(jnp.int32, (8, 128), 1)
    hit_c = jnp.logical_and(row8 == r2, lane8 == c)
    cnt_out[...] = cnt_in[...] + hit_c.astype(jnp.int32)

    lane256 = lax.broadcasted_iota(jnp.int32, (16 * _DIM, 128), 1)
    outer = vhi_in[...] * vlo_in[...]
    cov_out[...] = cov_in[...] + jnp.where(lane256 == c, outer, 0.0)


def kernel(ix, vec, sum_buf, counts, cov_buf):
    ix_arr = jnp.reshape(jnp.asarray(ix, jnp.int32), (1,))
    sum_t = jnp.transpose(sum_buf, (1, 0))
    cov_t = jnp.transpose(cov_buf, (1, 2, 0)).reshape(16 * _DIM, _N_WORDS)
    cpad = jnp.concatenate(
        [counts, jnp.zeros((_CPAD,), jnp.int32)]).reshape(_CNT_ROWS, 128)
    vecb = jnp.broadcast_to(vec.reshape(_DIM, 1), (_DIM, 128))
    vhi = jnp.broadcast_to(
        vec.reshape(_DIM, 1, 1), (_DIM, _DIM, 128)).reshape(16 * _DIM, 128)
    vlo = jnp.broadcast_to(
        vec.reshape(1, _DIM, 1), (_DIM, _DIM, 128)).reshape(16 * _DIM, 128)
    grid_spec = pltpu.PrefetchScalarGridSpec(
        num_scalar_prefetch=1,
        grid=(1,),
        in_specs=[
            pl.BlockSpec((_DIM, 128), lambda i, s: (0, 0)),
            pl.BlockSpec((16 * _DIM, 128), lambda i, s: (0, 0)),
            pl.BlockSpec((16 * _DIM, 128), lambda i, s: (0, 0)),
            pl.BlockSpec((_DIM, 128), lambda i, s: (0, s[0] // 128)),
            pl.BlockSpec((8, 128), lambda i, s: (s[0] // 1024, 0)),
            pl.BlockSpec((16 * _DIM, 128), lambda i, s: (0, s[0] // 128)),
        ],
        out_specs=[
            pl.BlockSpec((_DIM, 128), lambda i, s: (0, s[0] // 128)),
            pl.BlockSpec((8, 128), lambda i, s: (s[0] // 1024, 0)),
            pl.BlockSpec((16 * _DIM, 128), lambda i, s: (0, s[0] // 128)),
        ],
    )
    out = pl.pallas_call(
        _body,
        grid_spec=grid_spec,
        out_shape=[
            jax.ShapeDtypeStruct((_DIM, _N_WORDS), jnp.float32),
            jax.ShapeDtypeStruct((_CNT_ROWS, 128), jnp.int32),
            jax.ShapeDtypeStruct((16 * _DIM, _N_WORDS), jnp.float32),
        ],
        input_output_aliases={4: 0, 5: 1, 6: 2},
    )(ix_arr, vecb, vhi, vlo, sum_t, cpad, cov_t)
    return (jnp.transpose(out[0], (1, 0)),
            out[1].reshape(-1)[:_N_WORDS],
            jnp.transpose(out[2].reshape(_DIM, _DIM, _N_WORDS), (2, 0, 1)))
